# Initial kernel scaffold; baseline (speedup 1.0000x reference)
import jax
import jax.numpy as jnp
from jax.experimental import pallas as pl


def kernel(q):
    raise NotImplementedError("write your pallas kernel here")


# speedup vs baseline: 2.6072x; 2.6072x over previous
import jax
import jax.numpy as jnp
from jax import lax
from jax.experimental import pallas as pl
from jax.experimental.pallas import tpu as pltpu
from jax.experimental.pallas import tpu_sc as plsc

_T = 16384
_E = 64
_NC, _NS, _L = 2, 16, 16
_NW = _NC * _NS
_RPW = _T // _NW
_G = _RPW // _L
_CHUNK = _RPW * _E


def _sc_body(q_hbm, acc_hbm, hist_hbm, chunk, acc, hist):
    wid = lax.axis_index("s") * _NC + lax.axis_index("c")
    base = wid * _CHUNK
    pltpu.sync_copy(q_hbm.at[pl.ds(base, _CHUNK)], chunk)

    zf = jnp.zeros((_L,), jnp.float32)
    for e in range(_E):
        acc[pl.ds(e * _L, _L)] = zf
    for j in range(_E // _L):
        hist[pl.ds(j * _L, _L)] = zf

    ones = jnp.ones((_L,), jnp.float32)
    lane64 = lax.iota(jnp.int32, _L) * _E

    def g_body(g, carry):
        rows = g * (_L * _E) + lane64
        v = plsc.load_gather(chunk, [rows])
        acc[pl.ds(0, _L)] = acc[pl.ds(0, _L)] + v
        m = v
        a = jnp.zeros((_L,), jnp.int32)
        for e in range(1, _E):
            v = plsc.load_gather(chunk, [rows + e])
            acc[pl.ds(e * _L, _L)] = acc[pl.ds(e * _L, _L)] + v
            gt = v > m
            m = jnp.where(gt, v, m)
            a = jnp.where(gt, e, a)
        plsc.addupdate_scatter(hist, [a], ones)
        return carry

    lax.fori_loop(0, _G, g_body, 0)

    pltpu.sync_copy(acc, acc_hbm.at[wid])
    pltpu.sync_copy(hist, hist_hbm.at[wid])


_sc_call = pl.kernel(
    _sc_body,
    out_type=[
        jax.ShapeDtypeStruct((_NW, _E * _L), jnp.float32),
        jax.ShapeDtypeStruct((_NW, _E), jnp.float32),
    ],
    mesh=plsc.VectorSubcoreMesh(core_axis_name="c", subcore_axis_name="s"),
    compiler_params=pltpu.CompilerParams(needs_layout_passes=False),
    scratch_types=[
        pltpu.VMEM((_CHUNK,), jnp.float32),
        pltpu.VMEM((_E * _L,), jnp.float32),
        pltpu.VMEM((_E,), jnp.float32),
    ],
)


def _tc_body(a_ref, h_ref, o_ref):
    cs = jnp.sum(jnp.sum(a_ref[...], axis=0), axis=1)
    ct = jnp.sum(h_ref[...], axis=0)
    o_ref[...] = (jnp.sum(cs * ct) * (_E / (_T * _T))).reshape(1, 1)


def kernel(q):
    acc, hist = _sc_call(q.reshape(-1))
    out = pl.pallas_call(
        _tc_body,
        out_shape=jax.ShapeDtypeStruct((1, 1), jnp.float32),
    )(acc.reshape(_NW, _E, _L), hist)
    return out[0, 0]


# speedup vs baseline: 4.6591x; 1.7870x over previous
import jax
import jax.numpy as jnp
from jax import lax
from jax.experimental import pallas as pl
from jax.experimental.pallas import tpu as pltpu
from jax.experimental.pallas import tpu_sc as plsc

_T = 16384
_E = 64
_NC, _NS, _L = 2, 16, 16
_NW = _NC * _NS
_RPW = _T // _NW
_CHUNK = _RPW * _E
_UNROLL = 4


def _sc_body(q_hbm, out_hbm, chunk, obuf):
    wid = lax.axis_index("s") * _NC + lax.axis_index("c")
    base = wid * _CHUNK
    pltpu.sync_copy(q_hbm.at[pl.ds(base, _CHUNK)], chunk)

    pos = [lax.iota(jnp.int32, _L) + j * _L for j in range(4)]
    big = jnp.full((_L,), _E, jnp.int32)
    onef = jnp.ones((_L,), jnp.float32)
    zf = jnp.zeros((_L,), jnp.float32)

    def row_update(r, cs, hc):
        b = r * _E
        v = [chunk[pl.ds(b + j * _L, _L)] for j in range(4)]
        cs = [c + x for c, x in zip(cs, v)]
        m = jnp.maximum(jnp.maximum(v[0], v[1]), jnp.maximum(v[2], v[3]))
        gmax = lax.reduce_max(m, axes=(0,))
        cand = [jnp.where(x == gmax, p, big) for x, p in zip(v, pos)]
        cm = jnp.minimum(jnp.minimum(cand[0], cand[1]),
                         jnp.minimum(cand[2], cand[3]))
        first = lax.reduce_min(cm, axes=(0,))
        hc = [h + jnp.where(c == first, onef, zf) for h, c in zip(hc, cand)]
        return cs, hc

    def body(i, carry):
        cs = list(carry[:4])
        hc = list(carry[4:])
        for u in range(_UNROLL):
            cs, hc = row_update(i * _UNROLL + u, cs, hc)
        return tuple(cs) + tuple(hc)

    init = tuple(zf for _ in range(8))
    res = lax.fori_loop(0, _RPW // _UNROLL, body, init)

    for j in range(4):
        obuf[pl.ds(j * _L, _L)] = res[j]
        obuf[pl.ds(_E + j * _L, _L)] = res[4 + j]
    pltpu.sync_copy(obuf, out_hbm.at[wid])


_sc_call = pl.kernel(
    _sc_body,
    out_type=jax.ShapeDtypeStruct((_NW, 2 * _E), jnp.float32),
    mesh=plsc.VectorSubcoreMesh(core_axis_name="c", subcore_axis_name="s"),
    compiler_params=pltpu.CompilerParams(needs_layout_passes=False),
    scratch_types=[
        pltpu.VMEM((_CHUNK,), jnp.float32),
        pltpu.VMEM((2 * _E,), jnp.float32),
    ],
)


def _tc_body(p_ref, o_ref):
    s = jnp.sum(p_ref[...], axis=0)
    cs, ct = s[:_E], s[_E:]
    o_ref[...] = (jnp.sum(cs * ct) * (_E / (_T * _T))).reshape(1, 1)


def kernel(q):
    parts = _sc_call(q.reshape(-1))
    out = pl.pallas_call(
        _tc_body,
        out_shape=jax.ShapeDtypeStruct((1, 1), jnp.float32),
    )(parts)
    return out[0, 0]


# speedup vs baseline: 5.5908x; 1.2000x over previous
import jax
import jax.numpy as jnp
from jax import lax
from jax.experimental import pallas as pl
from jax.experimental.pallas import tpu as pltpu
from jax.experimental.pallas import tpu_sc as plsc

_T = 16384
_E = 64
_NC, _NS, _L = 2, 16, 16
_NW = _NC * _NS
_RPW = _T // _NW
_CHUNK = _RPW * _E
_UNROLL = 4


def _sc_body(q_hbm, out_hbm, chunk, obuf):
    wid = lax.axis_index("s") * _NC + lax.axis_index("c")
    base = wid * _RPW
    pltpu.sync_copy(q_hbm.at[pl.ds(base, _RPW)], chunk)

    pos = [lax.iota(jnp.int32, _L) + j * _L for j in range(4)]
    big = jnp.full((_L,), _E, jnp.int32)
    onef = jnp.ones((_L,), jnp.float32)
    zf = jnp.zeros((_L,), jnp.float32)

    def row_update(r, cs, hc):
        v = [chunk[r, pl.ds(j * _L, _L)] for j in range(4)]
        cs = [c + x for c, x in zip(cs, v)]
        m = jnp.maximum(jnp.maximum(v[0], v[1]), jnp.maximum(v[2], v[3]))
        gmax = lax.reduce_max(m, axes=(0,))
        cand = [jnp.where(x == gmax, p, big) for x, p in zip(v, pos)]
        cm = jnp.minimum(jnp.minimum(cand[0], cand[1]),
                         jnp.minimum(cand[2], cand[3]))
        first = lax.reduce_min(cm, axes=(0,))
        hc = [h + jnp.where(c == first, onef, zf) for h, c in zip(hc, cand)]
        return cs, hc

    def body(i, carry):
        cs = list(carry[:4])
        hc = list(carry[4:])
        for u in range(_UNROLL):
            cs, hc = row_update(i * _UNROLL + u, cs, hc)
        return tuple(cs) + tuple(hc)

    init = tuple(zf for _ in range(8))
    res = lax.fori_loop(0, _RPW // _UNROLL, body, init)

    for j in range(4):
        obuf[pl.ds(j * _L, _L)] = res[j]
        obuf[pl.ds(_E + j * _L, _L)] = res[4 + j]
    pltpu.sync_copy(obuf, out_hbm.at[wid])


_sc_call = pl.kernel(
    _sc_body,
    out_type=jax.ShapeDtypeStruct((_NW, 2 * _E), jnp.float32),
    mesh=plsc.VectorSubcoreMesh(core_axis_name="c", subcore_axis_name="s"),
    compiler_params=pltpu.CompilerParams(needs_layout_passes=False),
    scratch_types=[
        pltpu.VMEM((_RPW, _E), jnp.float32),
        pltpu.VMEM((2 * _E,), jnp.float32),
    ],
)


def _tc_body(p_ref, o_ref):
    s = jnp.sum(p_ref[...], axis=0)
    cs, ct = s[:_E], s[_E:]
    o_ref[...] = (jnp.sum(cs * ct) * (_E / (_T * _T))).reshape(1, 1)


def kernel(q):
    parts = _sc_call(q)
    out = pl.pallas_call(
        _tc_body,
        out_shape=jax.ShapeDtypeStruct((1, 1), jnp.float32),
    )(parts)
    return out[0, 0]
